# baseline (speedup 1.0000x reference)
import functools

import jax
import jax.numpy as jnp
from jax import lax
from jax.experimental import pallas as pl
from jax.experimental.pallas import tpu as pltpu
from jax.experimental.pallas import tpu_sc as plsc

_NUM_CORES = 2
_NUM_SUBCORES = 16
_LANES = 16
_NW = _NUM_CORES * _NUM_SUBCORES


@functools.lru_cache(maxsize=None)
def _make_sc_kernel(batch: int, n_sites: int):
    b_per_w = batch // _NW
    chunk = b_per_w * n_sites
    assert batch % (8 * _NW) == 0
    mesh = plsc.VectorSubcoreMesh(
        core_axis_name="c", subcore_axis_name="s")

    @functools.partial(
        pl.kernel,
        out_type=(
            jax.ShapeDtypeStruct((batch,), jnp.float32),
            jax.ShapeDtypeStruct((batch,), jnp.float32),
        ),
        mesh=mesh,
        compiler_params=pltpu.CompilerParams(needs_layout_passes=False),
        scratch_types=[
            pltpu.VMEM((chunk // 128, 128), jnp.int32),
            pltpu.VMEM((b_per_w,), jnp.int32),
            pltpu.VMEM((b_per_w,), jnp.float32),
            pltpu.VMEM((b_per_w,), jnp.float32),
            pltpu.SemaphoreType.DMA,
        ],
    )
    def sc_kernel(x_hbm, real_hbm, imag_hbm, out_r, out_i,
                  xv, idxv, rv, iv, sem):
        wid = lax.axis_index("s") * _NUM_CORES + lax.axis_index("c")
        base = wid * b_per_w
        rows_per_w = chunk // 128
        pltpu.sync_copy(x_hbm.at[pl.ds(wid * rows_per_w, rows_per_w), :], xv)

        lanes = lax.iota(jnp.int32, _LANES)
        zeros = jnp.zeros((_LANES,), jnp.int32)

        def body(i, carry):
            w = (i * _LANES + lanes) * n_sites
            acc = jnp.zeros((_LANES,), jnp.int32)
            for j in range(n_sites):
                xj = plsc.load_gather(xv, [zeros, w + j])
                acc = acc * 2 + ((1 - xj) >> 1)
            off = pl.multiple_of(i * _LANES, _LANES)
            idxv[pl.ds(off, _LANES)] = acc
            return carry

        lax.fori_loop(0, b_per_w // _LANES, body, 0)

        pltpu.async_copy(real_hbm.at[idxv], rv, sem).wait()
        pltpu.async_copy(imag_hbm.at[idxv], iv, sem).wait()
        pltpu.sync_copy(rv, out_r.at[pl.ds(base, b_per_w)])
        pltpu.sync_copy(iv, out_i.at[pl.ds(base, b_per_w)])

    return sc_kernel


def kernel(x, real, imag):
    batch, n_sites = x.shape
    x_rows = x.reshape(batch * n_sites // 128, 128)
    r, i = _make_sc_kernel(batch, n_sites)(x_rows, real, imag)
    return lax.complex(r, i)


# speedup vs baseline: 1.1028x; 1.1028x over previous
import functools

import jax
import jax.numpy as jnp
from jax import lax
from jax.experimental import pallas as pl
from jax.experimental.pallas import tpu as pltpu
from jax.experimental.pallas import tpu_sc as plsc

_NUM_CORES = 2
_NUM_SUBCORES = 16
_LANES = 16
_NW = _NUM_CORES * _NUM_SUBCORES


@functools.lru_cache(maxsize=None)
def _make_sc_kernel(batch: int, n_sites: int):
    b_per_w = batch // _NW
    chunk = b_per_w * n_sites
    assert batch % (8 * _NW) == 0
    mesh = plsc.VectorSubcoreMesh(
        core_axis_name="c", subcore_axis_name="s")

    @functools.partial(
        pl.kernel,
        out_type=(
            jax.ShapeDtypeStruct((batch,), jnp.float32),
            jax.ShapeDtypeStruct((batch,), jnp.float32),
        ),
        mesh=mesh,
        compiler_params=pltpu.CompilerParams(needs_layout_passes=False),
        scratch_types=[
            pltpu.VMEM((b_per_w, n_sites), jnp.int32),
            pltpu.VMEM((b_per_w,), jnp.int32),
            pltpu.VMEM((b_per_w,), jnp.float32),
            pltpu.VMEM((b_per_w,), jnp.float32),
            pltpu.SemaphoreType.DMA,
        ],
    )
    def sc_kernel(x_hbm, real_hbm, imag_hbm, out_r, out_i,
                  xv, idxv, rv, iv, sem):
        wid = lax.axis_index("s") * _NUM_CORES + lax.axis_index("c")
        base = wid * b_per_w
        pltpu.sync_copy(x_hbm.at[pl.ds(base, b_per_w), :], xv)

        lanes = lax.iota(jnp.int32, _LANES)
        last_lane = lanes == (_LANES - 1)
        one = jnp.ones((_LANES,), jnp.int32)
        w_hi = one << ((n_sites - 1) - lanes)
        tail_cols = jnp.minimum(lanes + (n_sites - 4), n_sites - 1)
        lo_mask = lanes < 4
        lo_amt = jnp.maximum(3 - lanes, 0)
        w_lo = jnp.where(lo_mask, one << lo_amt, 0)
        c_const = (1 << n_sites) - 1

        def body(i, carry):
            for k in range(_LANES):
                r = i * _LANES + k
                rfull = jnp.full((_LANES,), r, jnp.int32)
                v1 = xv[r, pl.ds(0, _LANES)]
                v2 = plsc.load_gather(xv, [rfull, tail_cols])
                t = plsc.cumsum(v1 * w_hi + v2 * w_lo)
                idx_vec = (c_const - t) >> 1
                plsc.store_scatter(idxv, [rfull], idx_vec, mask=last_lane)
            return carry

        lax.fori_loop(0, b_per_w // _LANES, body, 0)

        pltpu.async_copy(real_hbm.at[idxv], rv, sem).wait()
        pltpu.async_copy(imag_hbm.at[idxv], iv, sem).wait()
        pltpu.sync_copy(rv, out_r.at[pl.ds(base, b_per_w)])
        pltpu.sync_copy(iv, out_i.at[pl.ds(base, b_per_w)])

    return sc_kernel


def kernel(x, real, imag):
    batch, n_sites = x.shape
    r, i = _make_sc_kernel(batch, n_sites)(x, real, imag)
    return lax.complex(r, i)
